# baseline (speedup 1.0000x reference)
import functools

import jax
import jax.numpy as jnp
import numpy as np
from jax.experimental import pallas as pl
from jax.experimental.pallas import tpu as pltpu

_TINY = np.float32(1.1754943508222875e-38)


def _rotl(x, r):
    return (x << jnp.uint32(r)) | (x >> jnp.uint32(32 - r))


def _round4(x0, x1, rots):
    for r in rots:
        x0 = x0 + x1
        x1 = _rotl(x1, r) ^ x0
    return x0, x1


def _threefry_bits_pre(x1):
    ks1 = jnp.uint32(42)
    ks2 = jnp.uint32(0x1BD11BDA ^ 42)
    rot_a = (13, 15, 26, 6)
    rot_b = (17, 29, 16, 24)

    x0 = x1
    x1 = _rotl(x1, 13) ^ x0
    x0, x1 = _round4(x0, x1, rot_a[1:])
    x0 = x0 + ks1
    x1 = x1 + (ks2 + jnp.uint32(1))
    x0, x1 = _round4(x0, x1, rot_b)
    x0 = x0 + ks2
    x1 = x1 + jnp.uint32(2)
    x0, x1 = _round4(x0, x1, rot_a)
    x1 = x1 + (ks1 + jnp.uint32(3))
    x0, x1 = _round4(x0, x1, rot_b)
    x0 = x0 + ks1
    x1 = x1 + (ks2 + jnp.uint32(4))
    x0, x1 = _round4(x0, x1, rot_a)
    x0 = x0 + ks2
    x1 = x1 + jnp.uint32(5)
    return x0 ^ x1


def _body(vocab, n_chunks, width, lg_ref, out_ref, mx_ref, ix_ref):
    k = pl.program_id(1)
    rblk = pl.program_id(0)
    nb, chunk = lg_ref.shape
    n_sub = chunk // width

    last = k == n_chunks - 1
    limit = jnp.where(last, jnp.int32(vocab), jnp.int32(0x7FFFFFFF))

    row = jax.lax.broadcasted_iota(jnp.uint32, (nb, width), 0)
    colw = jax.lax.broadcasted_iota(jnp.uint32, (nb, width), 1)
    row0 = jnp.uint32(rblk) * jnp.uint32(nb)
    rowterm42 = row0 * jnp.uint32(vocab) + (row * jnp.uint32(vocab) + jnp.uint32(42))
    chunk0 = jnp.uint32(k) * jnp.uint32(chunk)

    def _sub(i, carry):
        mx, ix = carry
        base = chunk0 + jnp.uint32(i) * jnp.uint32(width)
        gcol_u = colw + base
        bits = _threefry_bits_pre(rowterm42 + gcol_u)
        fb = (bits >> jnp.uint32(9)) | jnp.uint32(0x3F800000)
        u = jax.lax.bitcast_convert_type(fb, jnp.float32) - jnp.float32(1.0)
        uni = jnp.maximum(u, _TINY)
        s = lg_ref[:, pl.ds(i * width, width)] - jnp.log(-jnp.log(uni))
        gcol = gcol_u.astype(jnp.int32)
        s = jnp.where(gcol < limit, s, -jnp.inf)
        better = s > mx
        mx = jnp.where(better, s, mx)
        ix = jnp.where(better, gcol, ix)
        return mx, ix

    mx0 = jnp.full((nb, width), -jnp.inf, jnp.float32)
    ix0 = jnp.zeros((nb, width), jnp.int32)
    mx, ix = jax.lax.fori_loop(0, n_sub, _sub, (mx0, ix0), unroll=64)

    @pl.when(k == 0)
    def _init():
        mx_ref[...] = mx
        ix_ref[...] = ix

    @pl.when(k > 0)
    def _merge():
        pmx = mx_ref[...]
        better = mx > pmx
        mx_ref[...] = jnp.where(better, mx, pmx)
        ix_ref[...] = jnp.where(better, ix, ix_ref[...])

    @pl.when(last)
    def _emit():
        fmx = mx_ref[...]
        m = jnp.max(fmx, axis=1, keepdims=True)
        cand = jnp.where(fmx == m, ix_ref[...], jnp.int32(0x7FFFFFFF))
        out_ref[...] = jnp.min(cand, axis=1, keepdims=True)


def kernel(logits):
    batch, vocab = logits.shape
    row_blk = 16
    chunk = 65536
    width = 256
    n_rblk = pl.cdiv(batch, row_blk)
    n_chunks = pl.cdiv(vocab, chunk)

    out = pl.pallas_call(
        functools.partial(_body, vocab, n_chunks, width),
        grid=(n_rblk, n_chunks),
        in_specs=[pl.BlockSpec((row_blk, chunk), lambda r, k: (r, k))],
        out_specs=pl.BlockSpec((row_blk, 1), lambda r, k: (r, 0)),
        out_shape=jax.ShapeDtypeStruct((batch, 1), jnp.int32),
        scratch_shapes=[
            pltpu.VMEM((row_blk, width), jnp.float32),
            pltpu.VMEM((row_blk, width), jnp.int32),
        ],
        compiler_params=pltpu.CompilerParams(
            dimension_semantics=("parallel", "arbitrary"),
        ),
    )(logits)
    return out.reshape(batch)


# speedup vs baseline: 1.0442x; 1.0442x over previous
import functools

import jax
import jax.numpy as jnp
import numpy as np
from jax.experimental import pallas as pl
from jax.experimental.pallas import tpu as pltpu

_TINY = np.float32(1.1754943508222875e-38)


def _rotl(x, r):
    return (x << jnp.uint32(r)) | (x >> jnp.uint32(32 - r))


def _round4(x0, x1, rots):
    for r in rots:
        x0 = x0 + x1
        x1 = _rotl(x1, r) ^ x0
    return x0, x1


def _threefry_bits_pre(x1):
    ks1 = jnp.uint32(42)
    ks2 = jnp.uint32(0x1BD11BDA ^ 42)
    rot_a = (13, 15, 26, 6)
    rot_b = (17, 29, 16, 24)

    x0 = x1
    x1 = _rotl(x1, 13) ^ x0
    x0, x1 = _round4(x0, x1, rot_a[1:])
    x0 = x0 + ks1
    x1 = x1 + (ks2 + jnp.uint32(1))
    x0, x1 = _round4(x0, x1, rot_b)
    x0 = x0 + ks2
    x1 = x1 + jnp.uint32(2)
    x0, x1 = _round4(x0, x1, rot_a)
    x1 = x1 + (ks1 + jnp.uint32(3))
    x0, x1 = _round4(x0, x1, rot_b)
    x0 = x0 + ks1
    x1 = x1 + (ks2 + jnp.uint32(4))
    x0, x1 = _round4(x0, x1, rot_a)
    x0 = x0 + ks2
    x1 = x1 + jnp.uint32(5)
    return x0 ^ x1


def _body(vocab, n_chunks, width, lg_ref, out_ref, mx_ref, ix_ref):
    k = pl.program_id(1)
    rblk = pl.program_id(0)
    nb, chunk = lg_ref.shape
    n_sub = chunk // width

    last = k == n_chunks - 1
    limit = jnp.where(last, jnp.int32(vocab), jnp.int32(0x7FFFFFFF))

    row = jax.lax.broadcasted_iota(jnp.uint32, (nb, width), 0)
    colw = jax.lax.broadcasted_iota(jnp.uint32, (nb, width), 1)
    row0 = jnp.uint32(rblk) * jnp.uint32(nb)
    rowterm42 = row0 * jnp.uint32(vocab) + (row * jnp.uint32(vocab) + jnp.uint32(42))
    chunk0 = jnp.uint32(k) * jnp.uint32(chunk)

    def _sub(i, carry):
        mx, ix = carry
        base = chunk0 + jnp.uint32(i) * jnp.uint32(width)
        gcol_u = colw + base
        bits = _threefry_bits_pre(rowterm42 + gcol_u)
        fb = (bits >> jnp.uint32(9)) | jnp.uint32(0x3F800000)
        u = jax.lax.bitcast_convert_type(fb, jnp.float32) - jnp.float32(1.0)
        uni = jnp.maximum(u, _TINY)
        s = lg_ref[:, pl.ds(i * width, width)] - jnp.log(-jnp.log(uni))
        gcol = gcol_u.astype(jnp.int32)
        s = jnp.where(gcol < limit, s, -jnp.inf)
        better = s > mx
        mx = jnp.where(better, s, mx)
        ix = jnp.where(better, gcol, ix)
        return mx, ix

    mx0 = jnp.full((nb, width), -jnp.inf, jnp.float32)
    ix0 = jnp.zeros((nb, width), jnp.int32)
    mx, ix = jax.lax.fori_loop(0, n_sub, _sub, (mx0, ix0), unroll=49)

    @pl.when(k == 0)
    def _init():
        mx_ref[...] = mx
        ix_ref[...] = ix

    @pl.when(k > 0)
    def _merge():
        pmx = mx_ref[...]
        better = mx > pmx
        mx_ref[...] = jnp.where(better, mx, pmx)
        ix_ref[...] = jnp.where(better, ix, ix_ref[...])

    @pl.when(last)
    def _emit():
        fmx = mx_ref[...]
        m = jnp.max(fmx, axis=1, keepdims=True)
        cand = jnp.where(fmx == m, ix_ref[...], jnp.int32(0x7FFFFFFF))
        out_ref[...] = jnp.min(cand, axis=1, keepdims=True)


def kernel(logits):
    batch, vocab = logits.shape
    row_blk = 16
    chunk = 50176
    width = 256
    n_rblk = pl.cdiv(batch, row_blk)
    n_chunks = pl.cdiv(vocab, chunk)

    out = pl.pallas_call(
        functools.partial(_body, vocab, n_chunks, width),
        grid=(n_rblk, n_chunks),
        in_specs=[pl.BlockSpec((row_blk, chunk), lambda r, k: (r, k))],
        out_specs=pl.BlockSpec((row_blk, 1), lambda r, k: (r, 0)),
        out_shape=jax.ShapeDtypeStruct((batch, 1), jnp.int32),
        scratch_shapes=[
            pltpu.VMEM((row_blk, width), jnp.float32),
            pltpu.VMEM((row_blk, width), jnp.int32),
        ],
        compiler_params=pltpu.CompilerParams(
            dimension_semantics=("parallel", "arbitrary"),
        ),
    )(logits)
    return out.reshape(batch)


# speedup vs baseline: 1.0554x; 1.0107x over previous
import functools

import jax
import jax.numpy as jnp
import numpy as np
from jax.experimental import pallas as pl
from jax.experimental.pallas import tpu as pltpu

_TINY = np.float32(1.1754943508222875e-38)


def _rotl(x, r):
    return (x << jnp.uint32(r)) | (x >> jnp.uint32(32 - r))


def _round4(x0, x1, rots):
    for r in rots:
        x0 = x0 + x1
        x1 = _rotl(x1, r) ^ x0
    return x0, x1


def _threefry_bits_pre(x1):
    ks1 = jnp.uint32(42)
    ks2 = jnp.uint32(0x1BD11BDA ^ 42)
    rot_a = (13, 15, 26, 6)
    rot_b = (17, 29, 16, 24)

    x0 = x1
    x1 = _rotl(x1, 13) ^ x0
    x0, x1 = _round4(x0, x1, rot_a[1:])
    x0 = x0 + ks1
    x1 = x1 + (ks2 + jnp.uint32(1))
    x0, x1 = _round4(x0, x1, rot_b)
    x0 = x0 + ks2
    x1 = x1 + jnp.uint32(2)
    x0, x1 = _round4(x0, x1, rot_a)
    x1 = x1 + (ks1 + jnp.uint32(3))
    x0, x1 = _round4(x0, x1, rot_b)
    x0 = x0 + ks1
    x1 = x1 + (ks2 + jnp.uint32(4))
    x0, x1 = _round4(x0, x1, rot_a)
    x0 = x0 + ks2
    x1 = x1 + jnp.uint32(5)
    return x0 ^ x1


def _body(vocab, n_chunks, width, lg_ref, out_ref, mx_ref, ix_ref):
    k = pl.program_id(1)
    rblk = pl.program_id(0)
    nb, chunk = lg_ref.shape
    n_sub = chunk // width

    last = k == n_chunks - 1
    limit = jnp.where(last, jnp.int32(vocab), jnp.int32(0x7FFFFFFF))

    row = jax.lax.broadcasted_iota(jnp.uint32, (nb, width), 0)
    colw = jax.lax.broadcasted_iota(jnp.uint32, (nb, width), 1)
    row0 = jnp.uint32(rblk) * jnp.uint32(nb)
    rowcol42 = (row0 * jnp.uint32(vocab)
                + (row * jnp.uint32(vocab) + jnp.uint32(42)) + colw)
    chunk0 = jnp.uint32(k) * jnp.uint32(chunk)
    id0 = k * n_sub

    def _make_sub(masked):
        def _sub(i, carry):
            mx, ix = carry
            base = chunk0 + jnp.uint32(i) * jnp.uint32(width)
            bits = _threefry_bits_pre(rowcol42 + base)
            fb = (bits >> jnp.uint32(9)) | jnp.uint32(0x3F800000)
            u = jax.lax.bitcast_convert_type(fb, jnp.float32) - jnp.float32(1.0)
            uni = jnp.maximum(u, _TINY)
            s = lg_ref[:, pl.ds(i * width, width)] - jnp.log(-jnp.log(uni))
            if masked:
                gcol = colw.astype(jnp.int32) + (k * chunk + i * width)
                s = jnp.where(gcol < limit, s, -jnp.inf)
            better = s > mx
            mx = jnp.where(better, s, mx)
            ix = jnp.where(better, id0 + i, ix)
            return mx, ix
        return _sub

    unroll = 49
    last_valid = vocab - (n_chunks - 1) * chunk
    vsafe = max(0, min(n_sub, last_valid // width)) // unroll * unroll
    mx0 = jnp.full((nb, width), -jnp.inf, jnp.float32)
    ix0 = jnp.zeros((nb, width), jnp.int32)
    mx, ix = jax.lax.fori_loop(0, vsafe, _make_sub(False), (mx0, ix0),
                               unroll=unroll)
    if vsafe < n_sub:
        mx, ix = jax.lax.fori_loop(vsafe, n_sub, _make_sub(True), (mx, ix),
                                   unroll=min(unroll, n_sub - vsafe))

    @pl.when(k == 0)
    def _init():
        mx_ref[...] = mx
        ix_ref[...] = ix

    @pl.when(k > 0)
    def _merge():
        pmx = mx_ref[...]
        better = mx > pmx
        mx_ref[...] = jnp.where(better, mx, pmx)
        ix_ref[...] = jnp.where(better, ix, ix_ref[...])

    @pl.when(last)
    def _emit():
        fmx = mx_ref[...]
        flat = ix_ref[...] * width + colw.astype(jnp.int32)
        m = jnp.max(fmx, axis=1, keepdims=True)
        cand = jnp.where(fmx == m, flat, jnp.int32(0x7FFFFFFF))
        out_ref[...] = jnp.min(cand, axis=1, keepdims=True)


def kernel(logits):
    batch, vocab = logits.shape
    row_blk = 16
    chunk = 50176
    width = 256
    n_rblk = pl.cdiv(batch, row_blk)
    n_chunks = pl.cdiv(vocab, chunk)

    out = pl.pallas_call(
        functools.partial(_body, vocab, n_chunks, width),
        grid=(n_rblk, n_chunks),
        in_specs=[pl.BlockSpec((row_blk, chunk), lambda r, k: (r, k))],
        out_specs=pl.BlockSpec((row_blk, 1), lambda r, k: (r, 0)),
        out_shape=jax.ShapeDtypeStruct((batch, 1), jnp.int32),
        scratch_shapes=[
            pltpu.VMEM((row_blk, width), jnp.float32),
            pltpu.VMEM((row_blk, width), jnp.int32),
        ],
        compiler_params=pltpu.CompilerParams(
            dimension_semantics=("parallel", "arbitrary"),
        ),
    )(logits)
    return out.reshape(batch)


# speedup vs baseline: 1.0584x; 1.0029x over previous
import functools

import jax
import jax.numpy as jnp
import numpy as np
from jax.experimental import pallas as pl
from jax.experimental.pallas import tpu as pltpu

_TINY = np.float32(1.1754943508222875e-38)


def _rotl(x, r):
    return (x << jnp.uint32(r)) | (x >> jnp.uint32(32 - r))


def _round4(x0, x1, rots):
    for r in rots:
        x0 = x0 + x1
        x1 = _rotl(x1, r) ^ x0
    return x0, x1


def _threefry_bits_pre(x1):
    ks1 = jnp.uint32(42)
    ks2 = jnp.uint32(0x1BD11BDA ^ 42)
    rot_a = (13, 15, 26, 6)
    rot_b = (17, 29, 16, 24)

    x0 = x1
    x1 = _rotl(x1, 13) ^ x0
    x0, x1 = _round4(x0, x1, rot_a[1:])
    x0 = x0 + ks1
    x1 = x1 + (ks2 + jnp.uint32(1))
    x0, x1 = _round4(x0, x1, rot_b)
    x0 = x0 + ks2
    x1 = x1 + jnp.uint32(2)
    x0, x1 = _round4(x0, x1, rot_a)
    x1 = x1 + (ks1 + jnp.uint32(3))
    x0, x1 = _round4(x0, x1, rot_b)
    x0 = x0 + ks1
    x1 = x1 + (ks2 + jnp.uint32(4))
    x0, x1 = _round4(x0, x1, rot_a)
    x0 = x0 + ks2
    x1 = x1 + jnp.uint32(5)
    return x0 ^ x1


def _body(vocab, n_chunks, width, lg_ref, out_ref, mx_ref, ix_ref):
    k = pl.program_id(1)
    rblk = pl.program_id(0)
    nb, chunk = lg_ref.shape
    n_sub = chunk // width

    last = k == n_chunks - 1
    limit = jnp.where(last, jnp.int32(vocab), jnp.int32(0x7FFFFFFF))

    row = jax.lax.broadcasted_iota(jnp.uint32, (nb, width), 0)
    colw = jax.lax.broadcasted_iota(jnp.uint32, (nb, width), 1)
    row0 = jnp.uint32(rblk) * jnp.uint32(nb)
    rowcol42 = (row0 * jnp.uint32(vocab)
                + (row * jnp.uint32(vocab) + jnp.uint32(42)) + colw)
    chunk0 = jnp.uint32(k) * jnp.uint32(chunk)
    id0 = k * n_sub

    def _make_sub(masked):
        def _sub(i, carry):
            mx, ix = carry
            base = chunk0 + jnp.uint32(i) * jnp.uint32(width)
            bits = _threefry_bits_pre(rowcol42 + base)
            fb = (bits >> jnp.uint32(9)) | jnp.uint32(0x3F800000)
            u = jax.lax.bitcast_convert_type(fb, jnp.float32) - jnp.float32(1.0)
            uni = jnp.maximum(u, _TINY)
            s = lg_ref[:, pl.ds(i * width, width)] - jnp.log(-jnp.log(uni))
            if masked:
                gcol = colw.astype(jnp.int32) + (k * chunk + i * width)
                s = jnp.where(gcol < limit, s, -jnp.inf)
            better = s > mx
            mx = jnp.where(better, s, mx)
            ix = jnp.where(better, id0 + i, ix)
            return mx, ix
        return _sub

    unroll = 49
    last_valid = vocab - (n_chunks - 1) * chunk
    vsafe = max(0, min(n_sub, last_valid // width)) // unroll * unroll
    mx0 = jnp.full((nb, width), -jnp.inf, jnp.float32)
    ix0 = jnp.zeros((nb, width), jnp.int32)
    mx, ix = jax.lax.fori_loop(0, vsafe, _make_sub(False), (mx0, ix0),
                               unroll=unroll)
    if vsafe < n_sub:
        mx, ix = jax.lax.fori_loop(vsafe, n_sub, _make_sub(True), (mx, ix),
                                   unroll=min(unroll, n_sub - vsafe))

    @pl.when(k == 0)
    def _init():
        mx_ref[...] = mx
        ix_ref[...] = ix

    @pl.when(k > 0)
    def _merge():
        pmx = mx_ref[...]
        better = mx > pmx
        mx_ref[...] = jnp.where(better, mx, pmx)
        ix_ref[...] = jnp.where(better, ix, ix_ref[...])

    @pl.when(last)
    def _emit():
        fmx = mx_ref[...]
        flat = ix_ref[...] * width + colw.astype(jnp.int32)
        m = jnp.max(fmx, axis=1, keepdims=True)
        cand = jnp.where(fmx == m, flat, jnp.int32(0x7FFFFFFF))
        out_ref[...] = jnp.min(cand, axis=1, keepdims=True)


def kernel(logits):
    batch, vocab = logits.shape
    row_blk = 16
    chunk = 100352
    width = 256
    n_rblk = pl.cdiv(batch, row_blk)
    n_chunks = pl.cdiv(vocab, chunk)

    out = pl.pallas_call(
        functools.partial(_body, vocab, n_chunks, width),
        grid=(n_rblk, n_chunks),
        in_specs=[pl.BlockSpec((row_blk, chunk), lambda r, k: (r, k))],
        out_specs=pl.BlockSpec((row_blk, 1), lambda r, k: (r, 0)),
        out_shape=jax.ShapeDtypeStruct((batch, 1), jnp.int32),
        scratch_shapes=[
            pltpu.VMEM((row_blk, width), jnp.float32),
            pltpu.VMEM((row_blk, width), jnp.int32),
        ],
        compiler_params=pltpu.CompilerParams(
            dimension_semantics=("parallel", "arbitrary"),
        ),
    )(logits)
    return out.reshape(batch)
